# baseline (speedup 1.0000x reference)
import functools

import jax
import jax.numpy as jnp
from jax import lax
from jax.experimental import pallas as pl
from jax.experimental.pallas import tpu as pltpu
from jax.experimental.pallas import tpu_sc as plsc

VOCAB = 1000000
BATCH = 16384
HIST = 50
EMBED_DIM = 64
NC, NS = 2, 16
NW = NC * NS
BB = 128
NUNIT = HIST * (BATCH // BB)
PER_W = NUNIT // NW
NPAIR = 512 * ((VOCAB + 1023) // 1024)


def _make_tc_detile():
    def body(t_ref, o_ref):
        x = t_ref[...]
        o_ref[:, 0:EMBED_DIM] = x[:, 0:512].T
        o_ref[:, EMBED_DIM:128] = x[:, 512:1024].T

    nblk = (VOCAB + 1023) // 1024
    return pl.pallas_call(
        body,
        grid=(nblk,),
        in_specs=[pl.BlockSpec((EMBED_DIM, 1024), lambda i: (0, i))],
        out_specs=pl.BlockSpec((512, 128), lambda i: (i, 0)),
        out_shape=jax.ShapeDtypeStruct((512 * nblk, 128), jnp.float32),
    )


def _make_sc_gather():
    mesh = plsc.VectorSubcoreMesh(
        core_axis_name="c", subcore_axis_name="s", num_cores=NC, num_subcores=NS
    )

    @functools.partial(
        pl.kernel,
        out_type=jax.ShapeDtypeStruct(
            (HIST, 8, BATCH // BB, 8, BB), jnp.float32
        ),
        mesh=mesh,
        compiler_params=pltpu.CompilerParams(
            use_tc_tiling_on_sc=False, needs_layout_passes=False
        ),
        scratch_types=[
            pltpu.VMEM((PER_W, BB), jnp.int32),
            pltpu.VMEM((PER_W, BB), jnp.int32),
            pltpu.VMEM((BB, 2 * EMBED_DIM), jnp.float32),
            pltpu.VMEM((BB, 2 * EMBED_DIM), jnp.float32),
            pltpu.VMEM((8, 8, BB), jnp.float32),
            pltpu.VMEM((8, 8, BB), jnp.float32),
            pltpu.SemaphoreType.DMA,
            pltpu.SemaphoreType.DMA,
            pltpu.SemaphoreType.DMA,
            pltpu.SemaphoreType.DMA,
        ],
    )
    def k(table_hbm, idx_hbm, out_hbm, idx_v, par_v, ga, gb, ta, tb,
          gsa, gsb, ssa, ssb):
        wid = lax.axis_index("s") * NC + lax.axis_index("c")
        u0 = wid * PER_W
        pltpu.sync_copy(idx_hbm.at[pl.ds(u0, PER_W)], idx_v)

        def prep(u, carry):
            for c in range(BB // 16):
                vec = idx_v[u, pl.ds(c * 16, 16)]
                idx_v[u, pl.ds(c * 16, 16)] = (
                    ((vec >> 10) << 9) | (vec & 511)
                )
                par_v[u, pl.ds(c * 16, 16)] = ((vec >> 9) & 1) << 6
            return carry

        lax.fori_loop(0, PER_W, prep, 0)

        def gather(j, buf, sem):
            return pltpu.async_copy(table_hbm.at[idx_v.at[j]], buf, sem)

        lane = jnp.arange(16, dtype=jnp.int32)
        _g_idx = {}
        _s_idx = {}
        for dk in range(EMBED_DIM // 16):
            for j in range(16):
                d = dk * 16 + ((lane + j) % 16)
                _g_idx[dk, j] = d
                _s_idx[dk, j] = (d // 8, d % 8)
        def transpose(ju, gbuf, tbuf):
            def body(bk, carry):
                row = bk * 16 + lane
                pv = par_v[ju, pl.ds(bk * 16, 16)]
                for dk in range(EMBED_DIM // 16):
                    for j in range(16):
                        v = plsc.load_gather(gbuf, [row, pv + _g_idx[dk, j]])
                        dgv, dsv = _s_idx[dk, j]
                        plsc.store_scatter(tbuf, [dgv, dsv, row], v)
                return carry

            lax.fori_loop(0, BB // 16, body, 0)

        def put(j, tbuf, sem):
            u = u0 + j
            h = u // (BATCH // BB)
            bt = u % (BATCH // BB)
            pltpu.async_copy(tbuf, out_hbm.at[h, :, bt], sem)

        def drain(tbuf, sem):
            pltpu.make_async_copy(tbuf, out_hbm.at[0, :, 0], sem).wait()

        def wait_g(buf, sem):
            pltpu.make_async_copy(table_hbm.at[pl.ds(0, BB)], buf, sem).wait()

        gather(0, ga, gsa)
        wait_g(ga, gsa)
        gather(1, gb, gsb)
        transpose(0, ga, ta)
        gather(2, ga, gsa)
        put(0, ta, ssa)
        wait_g(gb, gsb)
        transpose(1, gb, tb)
        gather(3, gb, gsb)
        put(1, tb, ssb)

        def body(i, carry):
            j1 = 2 * i + 2
            wait_g(ga, gsa)
            drain(ta, ssa)
            transpose(j1, ga, ta)

            @pl.when(j1 + 2 < PER_W)
            def _():
                gather(j1 + 2, ga, gsa)

            put(j1, ta, ssa)

            j2 = 2 * i + 3
            wait_g(gb, gsb)
            drain(tb, ssb)
            transpose(j2, gb, tb)

            @pl.when(j2 + 2 < PER_W)
            def _():
                gather(j2 + 2, gb, gsb)

            put(j2, tb, ssb)
            return carry

        lax.fori_loop(0, (PER_W - 2) // 2, body, 0)
        drain(ta, ssa)
        drain(tb, ssb)

    return k


_tc_detile = _make_tc_detile()
_sc_gather = _make_sc_gather()


@jax.jit
def kernel(x, table):
    idx = x.astype(jnp.int32).T.reshape(NUNIT, BB)
    table_pairs = _tc_detile(table.T)
    out5 = _sc_gather(table_pairs, idx)
    return out5.transpose(2, 4, 0, 1, 3).reshape(BATCH, HIST, EMBED_DIM)


# speedup vs baseline: 1.3959x; 1.3959x over previous
import functools

import jax
import jax.numpy as jnp
from jax import lax
from jax.experimental import pallas as pl
from jax.experimental.pallas import tpu as pltpu
from jax.experimental.pallas import tpu_sc as plsc

VOCAB = 1000000
BATCH = 16384
HIST = 50
EMBED_DIM = 64
NC, NS = 2, 16
NW = NC * NS
BB = 128
NUNIT = HIST * (BATCH // BB)
PER_W = NUNIT // NW
NPAIR = 2048 * ((VOCAB + 4095) // 4096)


def _make_tc_detile():
    def body(t_ref, o_ref):
        x = t_ref[...]
        for k in range(4):
            o_ref[pl.ds(512 * k, 512), 0:EMBED_DIM] = (
                x[:, 1024 * k : 1024 * k + 512].T
            )
            o_ref[pl.ds(512 * k, 512), EMBED_DIM:128] = (
                x[:, 1024 * k + 512 : 1024 * k + 1024].T
            )

    nblk = (VOCAB + 4095) // 4096
    return pl.pallas_call(
        body,
        grid=(nblk,),
        in_specs=[pl.BlockSpec((EMBED_DIM, 4096), lambda i: (0, i))],
        out_specs=pl.BlockSpec((2048, 128), lambda i: (i, 0)),
        out_shape=jax.ShapeDtypeStruct((2048 * nblk, 128), jnp.float32),
    )


def _make_sc_gather():
    mesh = plsc.VectorSubcoreMesh(
        core_axis_name="c", subcore_axis_name="s", num_cores=NC, num_subcores=NS
    )

    @functools.partial(
        pl.kernel,
        out_type=jax.ShapeDtypeStruct(
            (HIST, 8, BATCH // BB, 8, BB), jnp.float32
        ),
        mesh=mesh,
        compiler_params=pltpu.CompilerParams(
            use_tc_tiling_on_sc=False, needs_layout_passes=False
        ),
        scratch_types=[
            pltpu.VMEM((PER_W, BB), jnp.int32),
            pltpu.VMEM((BB, EMBED_DIM), jnp.float32),
            pltpu.VMEM((BB, EMBED_DIM), jnp.float32),
            pltpu.VMEM((8, 8, BB), jnp.float32),
            pltpu.VMEM((8, 8, BB), jnp.float32),
            pltpu.SemaphoreType.DMA,
            pltpu.SemaphoreType.DMA,
            pltpu.SemaphoreType.DMA,
            pltpu.SemaphoreType.DMA,
        ],
    )
    def k(table_hbm, idx_hbm, out_hbm, idx_v, ga, gb, ta, tb,
          gsa, gsb, ssa, ssb):
        wid = lax.axis_index("s") * NC + lax.axis_index("c")
        u0 = wid * PER_W
        pltpu.sync_copy(idx_hbm.at[pl.ds(u0, PER_W)], idx_v)

        def prep(u, carry):
            for c in range(BB // 16):
                vec = idx_v[u, pl.ds(c * 16, 16)]
                idx_v[u, pl.ds(c * 16, 16)] = (
                    ((vec >> 10) << 10) | ((vec & 511) << 1) | ((vec >> 9) & 1)
                )
            return carry

        lax.fori_loop(0, PER_W, prep, 0)

        def gather(j, buf, sem):
            return pltpu.async_copy(table_hbm.at[idx_v.at[j]], buf, sem)

        lane = jnp.arange(16, dtype=jnp.int32)
        _g_idx = {}
        _s_idx = {}
        for dk in range(EMBED_DIM // 16):
            for j in range(16):
                d = dk * 16 + ((lane + j) % 16)
                _g_idx[dk, j] = d
                _s_idx[dk, j] = (d // 8, d % 8)
        def transpose(gbuf, tbuf):
            def body(bk, carry):
                row = bk * 16 + lane
                for dk in range(EMBED_DIM // 16):
                    for j in range(16):
                        v = plsc.load_gather(gbuf, [row, _g_idx[dk, j]])
                        dgv, dsv = _s_idx[dk, j]
                        plsc.store_scatter(tbuf, [dgv, dsv, row], v)
                return carry

            lax.fori_loop(0, BB // 16, body, 0)

        def put(j, tbuf, sem):
            u = u0 + j
            h = u // (BATCH // BB)
            bt = u % (BATCH // BB)
            pltpu.async_copy(tbuf, out_hbm.at[h, :, bt], sem)

        def drain(tbuf, sem):
            pltpu.make_async_copy(tbuf, out_hbm.at[0, :, 0], sem).wait()

        def wait_g(buf, sem):
            pltpu.make_async_copy(table_hbm.at[pl.ds(0, BB)], buf, sem).wait()

        gather(0, ga, gsa)
        wait_g(ga, gsa)
        gather(1, gb, gsb)
        transpose(ga, ta)
        gather(2, ga, gsa)
        put(0, ta, ssa)
        wait_g(gb, gsb)
        transpose(gb, tb)
        gather(3, gb, gsb)
        put(1, tb, ssb)

        def body(i, carry):
            j1 = 2 * i + 2
            wait_g(ga, gsa)
            drain(ta, ssa)
            transpose(ga, ta)

            @pl.when(j1 + 2 < PER_W)
            def _():
                gather(j1 + 2, ga, gsa)

            put(j1, ta, ssa)

            j2 = 2 * i + 3
            wait_g(gb, gsb)
            drain(tb, ssb)
            transpose(gb, tb)

            @pl.when(j2 + 2 < PER_W)
            def _():
                gather(j2 + 2, gb, gsb)

            put(j2, tb, ssb)
            return carry

        lax.fori_loop(0, (PER_W - 2) // 2, body, 0)
        drain(ta, ssa)
        drain(tb, ssb)

    return k


_tc_detile = _make_tc_detile()
_sc_gather = _make_sc_gather()


@jax.jit
def kernel(x, table):
    idx = x.astype(jnp.int32).T.reshape(NUNIT, BB)
    table_rows = _tc_detile(table.T).reshape(2 * NPAIR, EMBED_DIM)
    out5 = _sc_gather(table_rows, idx)
    return out5.transpose(2, 4, 0, 1, 3).reshape(BATCH, HIST, EMBED_DIM)
